# Initial kernel scaffold; baseline (speedup 1.0000x reference)
import jax
import jax.numpy as jnp
from jax.experimental import pallas as pl


def kernel(hm_cen, box_preds, K):
    raise NotImplementedError("write your pallas kernel here")


# speedup vs baseline: 1.0189x; 1.0189x over previous
import jax
import jax.numpy as jnp
from jax.experimental import pallas as pl
from jax.experimental.pallas import tpu as pltpu

_H = 512
_W = 512
_NEG = float("-inf")


def _nms_body(x_ref, o_ref):
    x = x_ref[0, 0]
    s = jax.nn.sigmoid(x)
    l = jnp.concatenate([jnp.full((_H, 1), _NEG, jnp.float32), s[:, :-1]], axis=1)
    r = jnp.concatenate([s[:, 1:], jnp.full((_H, 1), _NEG, jnp.float32)], axis=1)
    h = jnp.maximum(jnp.maximum(l, r), s)
    u = jnp.concatenate([jnp.full((1, _W), _NEG, jnp.float32), h[:-1, :]], axis=0)
    d = jnp.concatenate([h[1:, :], jnp.full((1, _W), _NEG, jnp.float32)], axis=0)
    m = jnp.maximum(jnp.maximum(u, d), h)
    o_ref[0, 0] = jnp.where(m == s, s, 0.0)


def _nms_scores(hm_cen):
    B, C, H, W = hm_cen.shape
    return pl.pallas_call(
        _nms_body,
        grid=(B * C,),
        in_specs=[pl.BlockSpec((1, 1, H, W), lambda i: (i // C, i % C, 0, 0))],
        out_specs=pl.BlockSpec((1, 1, H, W), lambda i: (i // C, i % C, 0, 0)),
        out_shape=jax.ShapeDtypeStruct((B, C, H, W), jnp.float32),
    )(hm_cen)


def kernel(hm_cen, box_preds, K):
    B, C, H, W = hm_cen.shape
    Ks = 500
    scores_nms = _nms_scores(hm_cen)

    topk_scores, topk_inds = jax.lax.top_k(scores_nms.reshape(B, C, H * W), Ks)
    topk_score, topk_ind = jax.lax.top_k(topk_scores.reshape(B, C * Ks), Ks)
    clses = jnp.floor_divide(topk_ind, Ks).astype(jnp.float32)
    inds = jnp.take_along_axis(topk_inds.reshape(B, C * Ks), topk_ind, axis=1)
    row = jnp.floor_divide(inds, W).astype(jnp.float32)
    col = (inds % W).astype(jnp.float32)
    inds = inds + (jnp.asarray(K, dtype=inds.dtype) - Ks)

    feat = jnp.transpose(box_preds, (0, 2, 3, 1)).reshape(B, H * W, 8)
    g = jnp.take_along_axis(feat, inds[:, :, None], axis=1)
    det = jnp.concatenate([
        topk_score[:, :, None],
        col[:, :, None] + g[:, :, 0:1],
        row[:, :, None] + g[:, :, 1:2],
        g[:, :, 2:3],
        g[:, :, 3:6],
        g[:, :, 6:8],
        clses[:, :, None],
    ], axis=2)
    return det


# speedup vs baseline: 31.8950x; 31.3024x over previous
import functools

import jax
import jax.numpy as jnp
from jax import lax
from jax.experimental import pallas as pl
from jax.experimental.pallas import tpu as pltpu
from jax.experimental.pallas import tpu_sc as plsc

_H = 512
_W = 512
_NEG = float("-inf")
_KS = 500
_NW = 32
_CHUNK = 4096
_CAP = 128


def _nms_body(x_ref, o_ref, rm_ref):
    x = x_ref[0, 0]
    s = jax.nn.sigmoid(x)
    l = jnp.concatenate([jnp.full((_H, 1), _NEG, jnp.float32), s[:, :-1]], axis=1)
    r = jnp.concatenate([s[:, 1:], jnp.full((_H, 1), _NEG, jnp.float32)], axis=1)
    h = jnp.maximum(jnp.maximum(l, r), s)
    u = jnp.concatenate([jnp.full((1, _W), _NEG, jnp.float32), h[:-1, :]], axis=0)
    d = jnp.concatenate([h[1:, :], jnp.full((1, _W), _NEG, jnp.float32)], axis=0)
    m = jnp.maximum(jnp.maximum(u, d), h)
    masked = jnp.where(m == s, s, 0.0)
    o_ref[0, 0] = masked
    rm_ref[0, 0] = jnp.max(masked, axis=1)


def _nms_scores(hm_cen):
    B, C, H, W = hm_cen.shape
    return pl.pallas_call(
        _nms_body,
        grid=(B * C,),
        in_specs=[pl.BlockSpec((1, 1, H, W), lambda i: (i // C, i % C, 0, 0))],
        out_specs=[
            pl.BlockSpec((1, 1, H, W), lambda i: (i // C, i % C, 0, 0)),
            pl.BlockSpec((1, 1, H), lambda i: (i, 0, 0)),
        ],
        out_shape=[
            jax.ShapeDtypeStruct((B, C, H, W), jnp.float32),
            jax.ShapeDtypeStruct((B * C, 1, H), jnp.float32),
        ],
    )(hm_cen)


def _thresh_body(rm_ref, t_ref):
    bits = lax.bitcast_convert_type(rm_ref[...], jnp.int32)
    B = bits.shape[0]

    def step(_, carry):
        lo, hi = carry
        mid = lo + (hi - lo + 1) // 2
        cnt = jnp.sum((bits >= mid).astype(jnp.int32), axis=1, keepdims=True)
        ge = cnt >= _KS
        return jnp.where(ge, mid, lo), jnp.where(ge, hi, mid - 1)

    lo = jnp.zeros((B, 1), jnp.int32)
    hi = jnp.full((B, 1), 0x3F800000, jnp.int32)
    lo, hi = lax.fori_loop(0, 31, step, (lo, hi))
    t = lax.bitcast_convert_type(lo, jnp.float32)
    t_ref[...] = jnp.broadcast_to(t, t_ref.shape)


def _thresholds(rowmax_flat):
    B, R = rowmax_flat.shape
    return pl.pallas_call(
        _thresh_body,
        in_specs=[pl.BlockSpec((B, R), lambda: (0, 0))],
        out_specs=pl.BlockSpec((B, 16), lambda: (0, 0)),
        out_shape=jax.ShapeDtypeStruct((B, 16), jnp.float32),
    )(rowmax_flat)


_ROWS_PER_W = 1536 // _NW
_RB = _ROWS_PER_W * _W


def _compact_body(x_hbm, rm_hbm, th_hbm, outs_hbm, outi_hbm,
                  buf, rm_v, th_v, cs_v, ci_v, cnt_v, semA, semB):
    B = 4
    wid = lax.axis_index("s") * 2 + lax.axis_index("c")
    io = lax.iota(jnp.int32, 16)
    pltpu.sync_copy(th_hbm, th_v)
    sems = (semA, semB)
    hs = [pltpu.async_copy(x_hbm.at[pl.ds((b * _NW + wid) * _RB, _RB)],
                           buf.at[b % 2], sems[b % 2]) for b in range(2)]
    hs += [None, None]
    for b in range(B):
        pltpu.sync_copy(rm_hbm.at[pl.ds(b * 1536, 1536)], rm_v.at[pl.ds(0, 1536)])
        tv = th_v[pl.ds(b * 16, 16)]
        t_s = tv[0]
        for j in range(10):
            cs_v[pl.ds(j * 16, 16)] = jnp.full((16,), -1.0, jnp.float32)
            ci_v[pl.ds(j * 16, 16)] = jnp.zeros((16,), jnp.int32)
        cnt_v[pl.ds(0, 16)] = jnp.zeros((16,), jnp.int32)
        hs[b].wait()
        bsel = b % 2

        def rowloop(rl, dummy, bsel=bsel):
            row_abs = wid * _ROWS_PER_W + rl
            r_s = rm_v[pl.ds(row_abs, 16)][0]

            @pl.when(r_s >= t_s)
            def _():
                c0 = cnt_v[pl.ds(0, 16)][0]

                def vbody(i, cnt):
                    v = buf[bsel, pl.ds(rl * _W + i * 16, 16)]
                    m = v >= t_s
                    rank = jnp.where(m, jnp.int32(1), jnp.int32(0))
                    for sh in (1, 2, 4, 8):
                        rank = rank + jnp.where(
                            io >= sh, rank[jnp.maximum(io - sh, 0)], jnp.int32(0))
                    n = rank[15]

                    @pl.when(n > 0)
                    def _():
                        perm = jnp.zeros((16,), jnp.int32)
                        r_prev = jnp.int32(0)
                        for j in range(16):
                            r_j = rank[j]
                            tgt = jnp.where(r_j != r_prev, r_j - 1, jnp.int32(-1))
                            perm = perm + jnp.where(io == tgt, jnp.int32(j), jnp.int32(0))
                            r_prev = r_j
                        gi = (row_abs * _W + i * 16) + io
                        outv = jnp.where(io < n, v[perm], -1.0)
                        outi = jnp.where(io < n, gi[perm], jnp.int32(0))
                        cs_v[pl.ds(cnt, 16)] = outv
                        ci_v[pl.ds(cnt, 16)] = outi

                    return jnp.minimum(cnt + n, _CAP)

                c1 = lax.fori_loop(0, _W // 16, vbody, c0)
                cnt_v[pl.ds(0, 16)] = jnp.zeros((16,), jnp.int32) + c1

            return dummy

        lax.fori_loop(0, _ROWS_PER_W, rowloop, jnp.int32(0))
        if b + 2 < B:
            hs[b + 2] = pltpu.async_copy(
                x_hbm.at[pl.ds(((b + 2) * _NW + wid) * _RB, _RB)],
                buf.at[bsel], sems[bsel])
        obase = (b * _NW + wid) * _CAP
        pltpu.sync_copy(cs_v.at[pl.ds(0, _CAP)], outs_hbm.at[pl.ds(obase, _CAP)])
        pltpu.sync_copy(ci_v.at[pl.ds(0, _CAP)], outi_hbm.at[pl.ds(obase, _CAP)])


def _compact(masked_flat, rowmax_flat, th):
    B = 4
    mesh = plsc.VectorSubcoreMesh(core_axis_name="c", subcore_axis_name="s")
    f = functools.partial(
        pl.kernel,
        mesh=mesh,
        out_type=[
            jax.ShapeDtypeStruct((B * _NW * _CAP,), jnp.float32),
            jax.ShapeDtypeStruct((B * _NW * _CAP,), jnp.int32),
        ],
        scratch_types=[
            pltpu.VMEM((2, _RB), jnp.float32),
            pltpu.VMEM((1552,), jnp.float32),
            pltpu.VMEM((B * 16,), jnp.float32),
            pltpu.VMEM((160,), jnp.float32),
            pltpu.VMEM((160,), jnp.int32),
            pltpu.VMEM((16,), jnp.int32),
            pltpu.SemaphoreType.DMA,
            pltpu.SemaphoreType.DMA,
        ],
    )(_compact_body)
    return f(masked_flat, rowmax_flat, th)


def kernel(hm_cen, box_preds, K):
    B, C, H, W = hm_cen.shape
    masked, rowmax = _nms_scores(hm_cen)
    th = _thresholds(rowmax.reshape(B, C * H))
    cand_s, cand_i = _compact(masked.reshape(B * C * H * W),
                              rowmax.reshape(B * C * H), th.reshape(B * 16))
    cand_s = cand_s.reshape(B, _NW * _CAP)
    cand_i = cand_i.reshape(B, _NW * _CAP)

    topk_score, order = jax.lax.top_k(cand_s, _KS)
    inds = jnp.take_along_axis(cand_i, order, axis=1)
    clses = jnp.floor_divide(inds, H * W).astype(jnp.float32)
    inds = inds % (H * W)
    row = jnp.floor_divide(inds, W).astype(jnp.float32)
    col = (inds % W).astype(jnp.float32)
    inds = inds + (jnp.asarray(K, dtype=inds.dtype) - _KS)

    feat = jnp.transpose(box_preds, (0, 2, 3, 1)).reshape(B, H * W, 8)
    g = jnp.take_along_axis(feat, inds[:, :, None], axis=1)
    det = jnp.concatenate([
        topk_score[:, :, None],
        col[:, :, None] + g[:, :, 0:1],
        row[:, :, None] + g[:, :, 1:2],
        g[:, :, 2:3],
        g[:, :, 3:6],
        g[:, :, 6:8],
        clses[:, :, None],
    ], axis=2)
    return det


# speedup vs baseline: 32.1893x; 1.0092x over previous
import functools

import jax
import jax.numpy as jnp
from jax import lax
from jax.experimental import pallas as pl
from jax.experimental.pallas import tpu as pltpu
from jax.experimental.pallas import tpu_sc as plsc

_H = 512
_W = 512
_NEG = float("-inf")
_KS = 500
_NW = 32
_CHUNK = 4096
_CAP = 128


def _nms_body(x_ref, o_ref, rm_ref):
    x = x_ref[0, 0]
    s = jax.nn.sigmoid(x)
    l = jnp.concatenate([jnp.full((_H, 1), _NEG, jnp.float32), s[:, :-1]], axis=1)
    r = jnp.concatenate([s[:, 1:], jnp.full((_H, 1), _NEG, jnp.float32)], axis=1)
    h = jnp.maximum(jnp.maximum(l, r), s)
    u = jnp.concatenate([jnp.full((1, _W), _NEG, jnp.float32), h[:-1, :]], axis=0)
    d = jnp.concatenate([h[1:, :], jnp.full((1, _W), _NEG, jnp.float32)], axis=0)
    m = jnp.maximum(jnp.maximum(u, d), h)
    masked = jnp.where(m == s, s, 0.0)
    o_ref[0, 0] = masked
    rm_ref[0, 0] = jnp.max(masked, axis=1)


def _nms_scores(hm_cen):
    B, C, H, W = hm_cen.shape
    return pl.pallas_call(
        _nms_body,
        grid=(B * C,),
        in_specs=[pl.BlockSpec((1, 1, H, W), lambda i: (i // C, i % C, 0, 0))],
        out_specs=[
            pl.BlockSpec((1, 1, H, W), lambda i: (i // C, i % C, 0, 0)),
            pl.BlockSpec((1, 1, H), lambda i: (i, 0, 0)),
        ],
        out_shape=[
            jax.ShapeDtypeStruct((B, C, H, W), jnp.float32),
            jax.ShapeDtypeStruct((B * C, 1, H), jnp.float32),
        ],
    )(hm_cen)


def _thresh_body(rm_ref, t_ref):
    bits = lax.bitcast_convert_type(rm_ref[...], jnp.int32)
    B = bits.shape[0]

    def step(_, carry):
        lo, hi = carry
        mid = lo + (hi - lo + 1) // 2
        cnt = jnp.sum((bits >= mid).astype(jnp.int32), axis=1, keepdims=True)
        ge = cnt >= _KS
        return jnp.where(ge, mid, lo), jnp.where(ge, hi, mid - 1)

    lo = jnp.zeros((B, 1), jnp.int32)
    hi = jnp.full((B, 1), 0x3F800000, jnp.int32)
    lo, hi = lax.fori_loop(0, 31, step, (lo, hi))
    t = lax.bitcast_convert_type(lo, jnp.float32)
    t_ref[...] = jnp.broadcast_to(t, t_ref.shape)


def _thresholds(rowmax_flat):
    B, R = rowmax_flat.shape
    return pl.pallas_call(
        _thresh_body,
        in_specs=[pl.BlockSpec((B, R), lambda: (0, 0))],
        out_specs=pl.BlockSpec((B, 16), lambda: (0, 0)),
        out_shape=jax.ShapeDtypeStruct((B, 16), jnp.float32),
    )(rowmax_flat)


_ROWS_PER_W = 1536 // _NW
_RB = _ROWS_PER_W * _W


def _compact_body(x_hbm, rm_hbm, th_hbm, outs_hbm, outi_hbm,
                  buf, rm_v, th_v, cs_v, ci_v, cnt_v, semA, semB):
    B = 4
    wid = lax.axis_index("s") * 2 + lax.axis_index("c")
    io = lax.iota(jnp.int32, 16)
    pltpu.sync_copy(th_hbm, th_v)
    sems = (semA, semB)
    hs = [pltpu.async_copy(x_hbm.at[pl.ds((b * _NW + wid) * _RB, _RB)],
                           buf.at[b % 2], sems[b % 2]) for b in range(2)]
    hs += [None, None]
    for b in range(B):
        pltpu.sync_copy(rm_hbm.at[pl.ds(b * 1536, 1536)], rm_v.at[pl.ds(0, 1536)])
        tv = th_v[pl.ds(b * 16, 16)]
        t_s = tv[0]
        for j in range(10):
            cs_v[pl.ds(j * 16, 16)] = jnp.full((16,), -1.0, jnp.float32)
            ci_v[pl.ds(j * 16, 16)] = jnp.zeros((16,), jnp.int32)
        cnt_v[pl.ds(0, 16)] = jnp.zeros((16,), jnp.int32)
        hs[b].wait()
        bsel = b % 2

        def rowloop(rl, dummy, bsel=bsel):
            row_abs = wid * _ROWS_PER_W + rl
            r_s = rm_v[pl.ds(row_abs, 16)][0]

            @pl.when(r_s >= t_s)
            def _():
                c0 = cnt_v[pl.ds(0, 16)][0]

                def vbody(i, cnt):
                    v = buf[bsel, pl.ds(rl * _W + i * 16, 16)]
                    m = v >= t_s
                    rank = jnp.where(m, jnp.int32(1), jnp.int32(0))
                    for sh in (1, 2, 4, 8):
                        rank = rank + jnp.where(
                            io >= sh, rank[jnp.maximum(io - sh, 0)], jnp.int32(0))
                    n = rank[15]

                    @pl.when(n > 0)
                    def _():
                        perm = jnp.zeros((16,), jnp.int32)
                        r_prev = jnp.int32(0)
                        for j in range(16):
                            r_j = rank[j]
                            tgt = jnp.where(r_j != r_prev, r_j - 1, jnp.int32(-1))
                            perm = perm + jnp.where(io == tgt, jnp.int32(j), jnp.int32(0))
                            r_prev = r_j
                        gi = (row_abs * _W + i * 16) + io
                        outv = jnp.where(io < n, v[perm], -1.0)
                        outi = jnp.where(io < n, gi[perm], jnp.int32(0))
                        cs_v[pl.ds(cnt, 16)] = outv
                        ci_v[pl.ds(cnt, 16)] = outi

                    return jnp.minimum(cnt + n, _CAP)

                c1 = lax.fori_loop(0, _W // 16, vbody, c0)
                cnt_v[pl.ds(0, 16)] = jnp.zeros((16,), jnp.int32) + c1

            return dummy

        lax.fori_loop(0, _ROWS_PER_W, rowloop, jnp.int32(0))
        if b + 2 < B:
            hs[b + 2] = pltpu.async_copy(
                x_hbm.at[pl.ds(((b + 2) * _NW + wid) * _RB, _RB)],
                buf.at[bsel], sems[bsel])
        obase = (b * _NW + wid) * _CAP
        pltpu.sync_copy(cs_v.at[pl.ds(0, _CAP)], outs_hbm.at[pl.ds(obase, _CAP)])
        pltpu.sync_copy(ci_v.at[pl.ds(0, _CAP)], outi_hbm.at[pl.ds(obase, _CAP)])


def _compact(masked_flat, rowmax_flat, th):
    B = 4
    mesh = plsc.VectorSubcoreMesh(core_axis_name="c", subcore_axis_name="s")
    f = functools.partial(
        pl.kernel,
        mesh=mesh,
        out_type=[
            jax.ShapeDtypeStruct((B * _NW * _CAP,), jnp.float32),
            jax.ShapeDtypeStruct((B * _NW * _CAP,), jnp.int32),
        ],
        scratch_types=[
            pltpu.VMEM((2, _RB), jnp.float32),
            pltpu.VMEM((1552,), jnp.float32),
            pltpu.VMEM((B * 16,), jnp.float32),
            pltpu.VMEM((160,), jnp.float32),
            pltpu.VMEM((160,), jnp.int32),
            pltpu.VMEM((16,), jnp.int32),
            pltpu.SemaphoreType.DMA,
            pltpu.SemaphoreType.DMA,
        ],
    )(_compact_body)
    return f(masked_flat, rowmax_flat, th)


def _sort_body(s_ref, i_ref, os_ref, oi_ref, orc_ref):
    kb = lax.bitcast_convert_type(s_ref[...], jnp.int32)
    ix = i_ref[...]
    R, L = kb.shape
    lin = lax.broadcasted_iota(jnp.int32, (R, L), 0) * L + \
        lax.broadcasted_iota(jnp.int32, (R, L), 1)

    def ce(kb, ix, k, j, uniform):
        logj = j.bit_length() - 1
        logk = k.bit_length() - 1
        bitj = (lin >> logj) & 1
        if uniform:
            km_i = bitj ^ 1
        else:
            km_i = (((lin >> logk) ^ (lin >> logj)) & 1) ^ 1
        bj = bitj == 1
        if j < L:
            kp = jnp.where(bj, jnp.roll(kb, j, axis=1), jnp.roll(kb, -j, axis=1))
            ip = jnp.where(bj, jnp.roll(ix, j, axis=1), jnp.roll(ix, -j, axis=1))
        else:
            dr = j // L
            kp = jnp.where(bj, jnp.roll(kb, dr, axis=0), jnp.roll(kb, -dr, axis=0))
            ip = jnp.where(bj, jnp.roll(ix, dr, axis=0), jnp.roll(ix, -dr, axis=0))
        adv = (kp > kb) | ((kp == kb) & (ip < ix))
        adv_i = jnp.where(adv, jnp.int32(1), jnp.int32(0))
        take_p = (adv_i ^ km_i) == 0
        return jnp.where(take_p, kp, kb), jnp.where(take_p, ip, ix)

    N = R * L // 4
    k = 2
    while k <= N:
        j = k // 2
        while j >= 1:
            kb, ix = ce(kb, ix, k, j, k == N)
            j //= 2
        k *= 2
    os_ref[...] = lax.bitcast_convert_type(kb, jnp.float32)
    oi_ref[...] = ix
    colf = (ix % _W).astype(jnp.float32)
    rowf = ((ix // _W) % _H).astype(jnp.float32)
    clsf = (ix // (_H * _W)).astype(jnp.float32)
    orc_ref[0] = colf
    orc_ref[1] = rowf
    orc_ref[2] = clsf


def _sort4096(cand_s, cand_i):
    R, L = 128, 128
    return pl.pallas_call(
        _sort_body,
        in_specs=[pl.BlockSpec((R, L), lambda: (0, 0)),
                  pl.BlockSpec((R, L), lambda: (0, 0))],
        out_specs=[pl.BlockSpec((R, L), lambda: (0, 0)),
                   pl.BlockSpec((R, L), lambda: (0, 0)),
                   pl.BlockSpec((3, R, L), lambda: (0, 0, 0))],
        out_shape=[jax.ShapeDtypeStruct((R, L), jnp.float32),
                   jax.ShapeDtypeStruct((R, L), jnp.int32),
                   jax.ShapeDtypeStruct((3, R, L), jnp.float32)],
    )(cand_s.reshape(R, L), cand_i.reshape(R, L))


def _gather_body(idx_hbm, tab_hbm, out_hbm, idx_v, rows_v, gv, ov, sem):
    wid = lax.axis_index("s") * 2 + lax.axis_index("c")
    io = lax.iota(jnp.int32, 16)
    pltpu.sync_copy(idx_hbm.at[pl.ds(wid * 512, 512)], idx_v)

    def prep(v, d):
        rows_v[pl.ds(v * 16, 16)] = idx_v[pl.ds(v * 16, 16)] >> 7
        return d

    lax.fori_loop(0, 32, prep, jnp.int32(0))
    hs = [pltpu.async_copy(tab_hbm.at[rows_v.at[pl.ds(q * 128, 128)]],
                           gv.at[pl.ds(q * 128, 128)], sem) for q in range(4)]
    for h in hs:
        h.wait()
    for v in range(32):
        mm = idx_v[pl.ds(v * 16, 16)] & 127
        qv = mm >> 4
        lv = mm & 15
        res = jnp.zeros((16,), jnp.float32)
        for j in range(16):
            q_s = qv[j]
            w = gv[v * 16 + j, pl.ds(q_s * 16, 16)]
            lsp = lv[jnp.full((16,), j, jnp.int32)]
            res = jnp.where(io == j, w[lsp], res)
        ov[pl.ds(v * 16, 16)] = res
    pltpu.sync_copy(ov, out_hbm.at[pl.ds(wid * 512, 512)])


def _gather(idx_glob, box_tab):
    mesh = plsc.VectorSubcoreMesh(core_axis_name="c", subcore_axis_name="s")
    f = functools.partial(
        pl.kernel,
        mesh=mesh,
        out_type=jax.ShapeDtypeStruct((4 * 8 * 512,), jnp.float32),
        scratch_types=[
            pltpu.VMEM((512,), jnp.int32),
            pltpu.VMEM((512,), jnp.int32),
            pltpu.VMEM((512, 128), jnp.float32),
            pltpu.VMEM((512,), jnp.float32),
            pltpu.SemaphoreType.DMA,
        ],
    )(_gather_body)
    return f(idx_glob, box_tab)


def kernel(hm_cen, box_preds, K):
    B, C, H, W = hm_cen.shape
    masked, rowmax = _nms_scores(hm_cen)
    th = _thresholds(rowmax.reshape(B, C * H))
    cand_s, cand_i = _compact(masked.reshape(B * C * H * W),
                              rowmax.reshape(B * C * H), th.reshape(B * 16))
    s_sorted, i_sorted, rcc = _sort4096(cand_s, cand_i)
    score = s_sorted.reshape(B, 4096)[:, :_KS]
    iso = i_sorted.reshape(B, 4096)
    rcc = rcc.reshape(3, B, 4096)
    col = rcc[0][:, :_KS]
    rowf = rcc[1][:, :_KS]
    cls = rcc[2][:, :_KS]

    idx512 = iso[:, :512] % (H * W)
    koff = jnp.asarray(K, jnp.int32) - _KS
    chbase = (jnp.arange(B, dtype=jnp.int32)[:, None, None] * 8
              + jnp.arange(8, dtype=jnp.int32)[None, :, None]) * (H * W)
    idx_glob = idx512[:, None, :] + koff + chbase
    g = _gather(idx_glob.reshape(B * 8 * 512), box_preds.reshape(B * 8 * H * W // 128, 128))
    g = g.reshape(B, 8, 512)[:, :, :_KS]

    det = jnp.concatenate([
        score[:, :, None],
        (col + g[:, 0])[:, :, None],
        (rowf + g[:, 1])[:, :, None],
        g[:, 2][:, :, None],
        jnp.transpose(g[:, 3:6], (0, 2, 1)),
        jnp.transpose(g[:, 6:8], (0, 2, 1)),
        cls[:, :, None],
    ], axis=2)
    return det
